# Initial kernel scaffold; baseline (speedup 1.0000x reference)
import jax
import jax.numpy as jnp
from jax.experimental import pallas as pl


def kernel(atom_feat, bond_feat, edge_idx, v_W, v_b, fa1_W, fa1_b, fa2_W, fa2_b, fa3_W, fa3_b, conv_W, conv_b, bond_W, bond_b):
    raise NotImplementedError("write your pallas kernel here")


# speedup vs baseline: 11.9694x; 11.9694x over previous
import functools

import numpy as np
import jax
import jax.numpy as jnp
from jax import lax
from jax.experimental import pallas as pl
from jax.experimental.pallas import tpu as pltpu
from jax.experimental.pallas import tpu_sc as plsc

N, E, D, H = 10000, 320000, 128, 8
DH = D // H
VW = 32
UW = 144
NC, NS = 2, 16
NW = NC * NS
EPW = E // NW
SCB = 80
NPW = N // NS

_HI = lax.Precision.HIGHEST

_A_np = np.zeros((VW, UW), np.float32)
_B_np = np.zeros((H, UW), np.float32)
for _c in range(DH):
    for _h in range(H):
        _A_np[_c, _c * H + _h] = 1.0
        _B_np[_h, _c * H + _h] = 1.0
for _h in range(H):
    _A_np[DH, D + _h] = 1.0
    _B_np[_h, D + _h] = 1.0
_T_np = _B_np[:, :D].copy()


def _tc1_body(a_ref, w1d_ref, w1s_ref, vw_ref, vb_ref, pd_ref, ps_ref, v_ref):
    a = a_ref[...]
    pd_ref[...] = jnp.dot(a, w1d_ref[...], precision=_HI,
                          preferred_element_type=jnp.float32)
    ps_ref[...] = jnp.dot(a, w1s_ref[...], precision=_HI,
                          preferred_element_type=jnp.float32)
    v_ref[...] = jnp.dot(a, vw_ref[...], precision=_HI,
                         preferred_element_type=jnp.float32) + vb_ref[...]


def _tc1(atom, w1d, w1s, vwe, vbe, interpret=False):
    nb = 5
    blk = N // nb
    return pl.pallas_call(
        _tc1_body,
        grid=(nb,),
        in_specs=[
            pl.BlockSpec((blk, D), lambda i: (i, 0)),
            pl.BlockSpec((D, D), lambda i: (0, 0)),
            pl.BlockSpec((D, D), lambda i: (0, 0)),
            pl.BlockSpec((D, VW), lambda i: (0, 0)),
            pl.BlockSpec((1, VW), lambda i: (0, 0)),
        ],
        out_specs=[
            pl.BlockSpec((blk, D), lambda i: (i, 0)),
            pl.BlockSpec((blk, D), lambda i: (i, 0)),
            pl.BlockSpec((blk, VW), lambda i: (i, 0)),
        ],
        out_shape=[
            jax.ShapeDtypeStruct((N, D), jnp.float32),
            jax.ShapeDtypeStruct((N, D), jnp.float32),
            jax.ShapeDtypeStruct((N, VW), jnp.float32),
        ],
        interpret=interpret,
    )(atom, w1d, w1s, vwe, vbe)


def _tc3_body(gd_ref, gs_ref, bf_ref, w1c_ref, b1_ref, w2_ref, b2_ref,
              w3_ref, b3_ref, s_ref, gmax_ref):
    x = (gd_ref[...] + gs_ref[...] + b1_ref[...]
         + jnp.dot(bf_ref[...], w1c_ref[...],
                   preferred_element_type=jnp.float32))
    h = jnp.maximum(x, 0.0)
    h = jnp.maximum(
        jnp.dot(h, w2_ref[...], preferred_element_type=jnp.float32)
        + b2_ref[...], 0.0)
    sv = jnp.dot(h, w3_ref[...], precision=_HI,
                 preferred_element_type=jnp.float32) + b3_ref[...]
    s_ref[...] = sv
    m = jnp.max(sv, axis=0, keepdims=True)
    i = pl.program_id(0)

    @pl.when(i == 0)
    def _():
        gmax_ref[...] = m

    @pl.when(i != 0)
    def _():
        gmax_ref[...] = jnp.maximum(gmax_ref[...], m)


def _tc3(gd, gs, bf, w1c, b1, w2, b2, w3, b3, interpret=False):
    blk = 512
    nb = E // blk
    return pl.pallas_call(
        _tc3_body,
        grid=(nb,),
        in_specs=[
            pl.BlockSpec((blk, D), lambda i: (i, 0)),
            pl.BlockSpec((blk, D), lambda i: (i, 0)),
            pl.BlockSpec((blk, D), lambda i: (i, 0)),
            pl.BlockSpec((D, D), lambda i: (0, 0)),
            pl.BlockSpec((1, D), lambda i: (0, 0)),
            pl.BlockSpec((D, D), lambda i: (0, 0)),
            pl.BlockSpec((1, D), lambda i: (0, 0)),
            pl.BlockSpec((D, H), lambda i: (0, 0)),
            pl.BlockSpec((1, H), lambda i: (0, 0)),
        ],
        out_specs=[
            pl.BlockSpec((blk, H), lambda i: (i, 0)),
            pl.BlockSpec((1, H), lambda i: (0, 0)),
        ],
        out_shape=[
            jax.ShapeDtypeStruct((E, H), jnp.float32),
            jax.ShapeDtypeStruct((1, H), jnp.float32),
        ],
        interpret=interpret,
    )(gd, gs, bf, w1c, b1, w2, b2, w3, b3)


def _tc4_body(s_ref, vd_ref, gmax_ref, a_ref, b_ref, um_ref):
    ex = jnp.exp(s_ref[...] - gmax_ref[...])
    vr = jnp.dot(vd_ref[...], a_ref[...], precision=_HI,
                 preferred_element_type=jnp.float32)
    er = jnp.dot(ex, b_ref[...], precision=_HI,
                 preferred_element_type=jnp.float32)
    um_ref[...] = vr * er


def _tc4(s, vd, gmax, amat, bmat, interpret=False):
    blk = 512
    nb = E // blk
    return pl.pallas_call(
        _tc4_body,
        grid=(nb,),
        in_specs=[
            pl.BlockSpec((blk, H), lambda i: (i, 0)),
            pl.BlockSpec((blk, VW), lambda i: (i, 0)),
            pl.BlockSpec((1, H), lambda i: (0, 0)),
            pl.BlockSpec((VW, UW), lambda i: (0, 0)),
            pl.BlockSpec((H, UW), lambda i: (0, 0)),
        ],
        out_specs=pl.BlockSpec((blk, UW), lambda i: (i, 0)),
        out_shape=jax.ShapeDtypeStruct((E, UW), jnp.float32),
        interpret=interpret,
    )(s, vd, gmax, amat, bmat)


def _tc6_body(pa_ref, pb_ref, atom_ref, convw_ref, convb_ref, wb1_ref, t_ref,
              out_ref, q_ref):
    tot = pa_ref[0] + pb_ref[0]
    un = tot[:, :D]
    den = tot[:, D:D + H]
    denb = jnp.dot(den, t_ref[...], precision=_HI,
                   preferred_element_type=jnp.float32)
    safe = jnp.where(denb > 0.0, denb, 1.0)
    ouf = un / safe
    out0 = jnp.dot(ouf, convw_ref[...], precision=_HI,
                   preferred_element_type=jnp.float32) + convb_ref[...]
    out_ref[...] = jnp.maximum(atom_ref[...] + out0, 0.0)
    q_ref[...] = jnp.dot(out0, wb1_ref[...], precision=_HI,
                         preferred_element_type=jnp.float32)


def _tc6(outp, atom, convw, convb, wb1, tmat, interpret=False):
    nb = 5
    blk = N // nb
    return pl.pallas_call(
        _tc6_body,
        grid=(nb,),
        in_specs=[
            pl.BlockSpec((1, blk, UW), lambda i: (0, i, 0)),
            pl.BlockSpec((1, blk, UW), lambda i: (1, i, 0)),
            pl.BlockSpec((blk, D), lambda i: (i, 0)),
            pl.BlockSpec((D, D), lambda i: (0, 0)),
            pl.BlockSpec((1, D), lambda i: (0, 0)),
            pl.BlockSpec((D, D), lambda i: (0, 0)),
            pl.BlockSpec((H, D), lambda i: (0, 0)),
        ],
        out_specs=[
            pl.BlockSpec((blk, D), lambda i: (i, 0)),
            pl.BlockSpec((blk, D), lambda i: (i, 0)),
        ],
        out_shape=[
            jax.ShapeDtypeStruct((N, D), jnp.float32),
            jax.ShapeDtypeStruct((N, D), jnp.float32),
        ],
        interpret=interpret,
    )(outp, outp, atom, convw, convb, wb1, tmat)


def _tc8_body(qd_ref, qs_ref, bf_ref, wb2_ref, bb_ref, nb_ref):
    acc = (qd_ref[...] + qs_ref[...] + bb_ref[...]
           + jnp.dot(bf_ref[...], wb2_ref[...],
                     preferred_element_type=jnp.float32))
    nb_ref[...] = jnp.maximum(acc, 0.0)


def _tc8(qd, qs, bf, wb2, bb, interpret=False):
    blk = 512
    nb = E // blk
    return pl.pallas_call(
        _tc8_body,
        grid=(nb,),
        in_specs=[
            pl.BlockSpec((blk, D), lambda i: (i, 0)),
            pl.BlockSpec((blk, D), lambda i: (i, 0)),
            pl.BlockSpec((blk, D), lambda i: (i, 0)),
            pl.BlockSpec((D, D), lambda i: (0, 0)),
            pl.BlockSpec((1, D), lambda i: (0, 0)),
        ],
        out_specs=pl.BlockSpec((blk, D), lambda i: (i, 0)),
        out_shape=jax.ShapeDtypeStruct((E, D), jnp.float32),
        interpret=interpret,
    )(qd, qs, bf, wb2, bb)


def _sc_mesh():
    return plsc.VectorSubcoreMesh(core_axis_name="c", subcore_axis_name="s",
                                  num_cores=NC, num_subcores=NS)


def _sc_gather3(ta, tb, tv, dst, src, interpret=False):
    def body(ta_ref, tb_ref, tv_ref, dst_ref, src_ref, gd_ref, gs_ref, vd_ref,
             idx_d, idx_s, buf_a, buf_b, buf_v, sem_a, sem_b, sem_v):
        c = lax.axis_index("c")
        s = lax.axis_index("s")
        w = c * NS + s
        base = w * EPW

        def step(t, carry):
            off = base + t * SCB
            pltpu.sync_copy(dst_ref.at[pl.ds(off, SCB)], idx_d)
            pltpu.sync_copy(src_ref.at[pl.ds(off, SCB)], idx_s)
            cp_a = pltpu.async_copy(ta_ref.at[idx_d], buf_a, sem_a)
            cp_b = pltpu.async_copy(tb_ref.at[idx_s], buf_b, sem_b)
            cp_v = pltpu.async_copy(tv_ref.at[idx_d], buf_v, sem_v)
            cp_a.wait()
            pltpu.sync_copy(buf_a, gd_ref.at[pl.ds(off, SCB)])
            cp_b.wait()
            pltpu.sync_copy(buf_b, gs_ref.at[pl.ds(off, SCB)])
            cp_v.wait()
            pltpu.sync_copy(buf_v, vd_ref.at[pl.ds(off, SCB)])
            return carry

        lax.fori_loop(0, EPW // SCB, step, 0)

    f = pl.kernel(
        body,
        out_type=[
            jax.ShapeDtypeStruct((E, D), jnp.float32),
            jax.ShapeDtypeStruct((E, D), jnp.float32),
            jax.ShapeDtypeStruct((E, VW), jnp.float32),
        ],
        mesh=_sc_mesh(),
        scratch_types=[
            pltpu.VMEM((SCB,), jnp.int32),
            pltpu.VMEM((SCB,), jnp.int32),
            pltpu.VMEM((SCB, D), jnp.float32),
            pltpu.VMEM((SCB, D), jnp.float32),
            pltpu.VMEM((SCB, VW), jnp.float32),
            pltpu.SemaphoreType.DMA,
            pltpu.SemaphoreType.DMA,
            pltpu.SemaphoreType.DMA,
        ],
        compiler_params=pltpu.CompilerParams(use_tc_tiling_on_sc=False),
        interpret=interpret,
    )
    return f(ta, tb, tv, dst, src)


def _sc_gather2(tq, dst, src, interpret=False):
    def body(tq_ref, dst_ref, src_ref, qd_ref, qs_ref,
             idx_d, idx_s, buf_a, buf_b, sem_a, sem_b):
        c = lax.axis_index("c")
        s = lax.axis_index("s")
        w = c * NS + s
        base = w * EPW

        def step(t, carry):
            off = base + t * SCB
            pltpu.sync_copy(dst_ref.at[pl.ds(off, SCB)], idx_d)
            pltpu.sync_copy(src_ref.at[pl.ds(off, SCB)], idx_s)
            cp_a = pltpu.async_copy(tq_ref.at[idx_d], buf_a, sem_a)
            cp_b = pltpu.async_copy(tq_ref.at[idx_s], buf_b, sem_b)
            cp_a.wait()
            pltpu.sync_copy(buf_a, qd_ref.at[pl.ds(off, SCB)])
            cp_b.wait()
            pltpu.sync_copy(buf_b, qs_ref.at[pl.ds(off, SCB)])
            return carry

        lax.fori_loop(0, EPW // SCB, step, 0)

    f = pl.kernel(
        body,
        out_type=[
            jax.ShapeDtypeStruct((E, D), jnp.float32),
            jax.ShapeDtypeStruct((E, D), jnp.float32),
        ],
        mesh=_sc_mesh(),
        scratch_types=[
            pltpu.VMEM((SCB,), jnp.int32),
            pltpu.VMEM((SCB,), jnp.int32),
            pltpu.VMEM((SCB, D), jnp.float32),
            pltpu.VMEM((SCB, D), jnp.float32),
            pltpu.SemaphoreType.DMA,
            pltpu.SemaphoreType.DMA,
        ],
        compiler_params=pltpu.CompilerParams(use_tc_tiling_on_sc=False),
        interpret=interpret,
    )
    return f(tq, dst, src)


def _sc_scatter(um, src, interpret=False):
    def body(um_ref, src_ref, outp_ref, idx_b, rows, acc, sem):
        c = lax.axis_index("c")
        s = lax.axis_index("s")

        def zrow(r, carry):
            for j in range(UW // 16):
                rows[r, pl.ds(j * 16, 16)] = jnp.zeros((16,), jnp.float32)
            return carry

        lax.fori_loop(0, SCB, zrow, 0)
        full, rem = divmod(NPW, SCB)
        for k in range(full):
            pltpu.sync_copy(rows, acc.at[pl.ds(s * NPW + k * SCB, SCB)])
        if rem:
            pltpu.sync_copy(rows.at[pl.ds(0, rem)],
                            acc.at[pl.ds(s * NPW + full * SCB, rem)])
        plsc.subcore_barrier()

        w = c * NS + s
        base = w * EPW

        def step(t, carry):
            off = base + t * SCB
            pltpu.sync_copy(src_ref.at[pl.ds(off, SCB)], idx_b)
            pltpu.sync_copy(um_ref.at[pl.ds(off, SCB)], rows)
            pltpu.sync_copy(rows, acc.at[idx_b], add=True)
            return carry

        lax.fori_loop(0, EPW // SCB, step, 0)
        plsc.subcore_barrier()
        pltpu.sync_copy(acc.at[pl.ds(s * NPW, NPW)],
                        outp_ref.at[c, pl.ds(s * NPW, NPW)])

    f = pl.kernel(
        body,
        out_type=jax.ShapeDtypeStruct((NC, N, UW), jnp.float32),
        mesh=_sc_mesh(),
        scratch_types=[
            pltpu.VMEM((SCB,), jnp.int32),
            pltpu.VMEM((SCB, UW), jnp.float32),
            pltpu.VMEM_SHARED((N, UW), jnp.float32),
            pltpu.SemaphoreType.DMA,
        ],
        compiler_params=pltpu.CompilerParams(use_tc_tiling_on_sc=False),
        interpret=interpret,
    )
    return f(um, src)


def _impl(atom_feat, bond_feat, edge_idx, v_W, v_b, fa1_W, fa1_b, fa2_W,
          fa2_b, fa3_W, fa3_b, conv_W, conv_b, bond_W, bond_b,
          interpret=False):
    src = edge_idx[:, 0]
    dst = edge_idx[:, 1]
    w1d = fa1_W[:D]
    w1s = fa1_W[D:2 * D]
    w1c = fa1_W[2 * D:]
    vwe = jnp.pad(v_W, ((0, 0), (0, VW - DH)))
    vbe = jnp.concatenate(
        [v_b, jnp.ones((1,), jnp.float32),
         jnp.zeros((VW - DH - 1,), jnp.float32)]).reshape(1, VW)
    b1 = fa1_b.reshape(1, D)
    b2 = fa2_b.reshape(1, D)
    b3 = fa3_b.reshape(1, H)
    convb = conv_b.reshape(1, D)
    bb = bond_b.reshape(1, D)
    amat = jnp.asarray(_A_np)
    bmat = jnp.asarray(_B_np)
    tmat = jnp.asarray(_T_np)

    pd, ps, ve = _tc1(atom_feat, w1d, w1s, vwe, vbe, interpret=interpret)
    gd, gs, vd = _sc_gather3(pd, ps, ve, dst, src, interpret=interpret)
    s, gmax = _tc3(gd, gs, bond_feat, w1c, b1, fa2_W, b2, fa3_W, b3,
                   interpret=interpret)
    um = _tc4(s, vd, gmax, amat, bmat, interpret=interpret)
    outp = _sc_scatter(um, src, interpret=interpret)
    out, q = _tc6(outp, atom_feat, conv_W, convb, bond_W[:D], tmat,
                  interpret=interpret)
    qd, qs = _sc_gather2(q, dst, src, interpret=interpret)
    new_bond = _tc8(qd, qs, bond_feat, bond_W[D:], bb, interpret=interpret)
    return out, new_bond


@jax.jit
def kernel(atom_feat, bond_feat, edge_idx, v_W, v_b, fa1_W, fa1_b, fa2_W,
           fa2_b, fa3_W, fa3_b, conv_W, conv_b, bond_W, bond_b):
    return _impl(atom_feat, bond_feat, edge_idx, v_W, v_b, fa1_W, fa1_b,
                 fa2_W, fa2_b, fa3_W, fa3_b, conv_W, conv_b, bond_W, bond_b)
